# Initial kernel scaffold; baseline (speedup 1.0000x reference)
import functools

import jax
import jax.numpy as jnp
from jax import lax
from jax.experimental import pallas as pl
from jax.experimental.pallas import tpu as pltpu
from jax.experimental.pallas import tpu_sc as plsc

_LAYER_DIMS = [128, 64, 32, 64, 128, 64, 128, 256, 128, 256]
_NUM_GRAPHS = 64
_NC = 2
_NS = 16
_NW = _NC * _NS
_CHUNK = 128
_DEG_W = 16


def _col_blocks(d):
    if d <= 128:
        return [d]
    assert d % 128 == 0
    return [128] * (d // 128)


def _row_chunks(total, step=128):
    out, off = [], 0
    while off < total:
        sz = min(step, total - off)
        out.append((off, sz))
        off += sz
    return out


def _fill_f32(buf, value):
    rows, w = buf.shape
    per_row = w // 16
    vec = jnp.full((16,), value, jnp.float32)

    def body(i, _):
        r = i // per_row
        c = i % per_row
        buf[r, pl.ds(c * 16, 16)] = vec
        return 0

    lax.fori_loop(0, rows * per_row, body, 0)


def _zero_spmem_rows(acc, zbuf, r0, nrows):
    for off, sz in _row_chunks(nrows):
        pltpu.sync_copy(zbuf.at[pl.ds(0, sz)], acc.at[pl.ds(r0 + off, sz)])


def _sc_degree(dst_p, npad, nch):
    rows_per_tile = npad // _NS
    mesh = plsc.VectorSubcoreMesh(core_axis_name="c", subcore_axis_name="s")

    @functools.partial(
        pl.kernel,
        out_type=jax.ShapeDtypeStruct((_NC, npad, _DEG_W), jnp.float32),
        mesh=mesh,
        scratch_types=[
            pltpu.VMEM((nch, _CHUNK), jnp.int32),
            pltpu.VMEM((_CHUNK, _DEG_W), jnp.float32),
            pltpu.VMEM((_CHUNK, _DEG_W), jnp.float32),
            pltpu.VMEM_SHARED((npad, _DEG_W), jnp.float32),
        ],
    )
    def deg_kernel(dst_hbm, out_hbm, idx_d, ones_v, zbuf, acc):
        c = lax.axis_index("c")
        s = lax.axis_index("s")
        wid = c * _NS + s
        r0 = s * rows_per_tile
        _fill_f32(ones_v, 1.0)
        _fill_f32(zbuf, 0.0)
        _zero_spmem_rows(acc, zbuf, r0, rows_per_tile)
        pltpu.sync_copy(dst_hbm.at[wid], idx_d)
        plsc.subcore_barrier()

        def body(j, _):
            pltpu.sync_copy(ones_v, acc.at[idx_d.at[j]], add=True)
            return 0

        lax.fori_loop(0, nch, body, 0)
        plsc.subcore_barrier()
        pltpu.sync_copy(acc.at[pl.ds(r0, rows_per_tile)],
                        out_hbm.at[c, pl.ds(r0, rows_per_tile)])

    return deg_kernel(dst_p)


def _sc_propagate(p_blocks, src_p, dst_p, npad, nch):
    nb = len(p_blocks)
    w = p_blocks[0].shape[1]
    rows_per_tile = npad // _NS
    mesh = plsc.VectorSubcoreMesh(core_axis_name="c", subcore_axis_name="s")

    @functools.partial(
        pl.kernel,
        out_type=jax.ShapeDtypeStruct((_NC, nb, npad, w), jnp.float32),
        mesh=mesh,
        scratch_types=[
            pltpu.VMEM((nch, _CHUNK), jnp.int32),
            pltpu.VMEM((nch, _CHUNK), jnp.int32),
            pltpu.VMEM((_CHUNK, w), jnp.float32),
            pltpu.VMEM((_CHUNK, w), jnp.float32),
            pltpu.VMEM_SHARED((npad, w), jnp.float32),
            pltpu.SemaphoreType.DMA,
        ],
    )
    def prop_kernel(*refs):
        p_refs = refs[:nb]
        src_hbm, dst_hbm, out_hbm = refs[nb:nb + 3]
        idx_s, idx_d, rows, zbuf, acc, sem = refs[nb + 3:]
        c = lax.axis_index("c")
        s = lax.axis_index("s")
        wid = c * _NS + s
        r0 = s * rows_per_tile
        _fill_f32(zbuf, 0.0)
        pltpu.sync_copy(src_hbm.at[wid], idx_s)
        pltpu.sync_copy(dst_hbm.at[wid], idx_d)
        for blk in range(nb):
            _zero_spmem_rows(acc, zbuf, r0, rows_per_tile)
            plsc.subcore_barrier()

            def body(j, _, blk=blk):
                pltpu.async_copy(p_refs[blk].at[idx_s.at[j]], rows, sem).wait()
                pltpu.sync_copy(rows, acc.at[idx_d.at[j]], add=True)
                return 0

            lax.fori_loop(0, nch, body, 0)
            plsc.subcore_barrier()
            pltpu.sync_copy(acc.at[pl.ds(r0, rows_per_tile)],
                            out_hbm.at[c, blk, pl.ds(r0, rows_per_tile)])

    return prop_kernel(*p_blocks, src_p, dst_p)


def _tc_init(deg_partials, xp, w0, npad):
    d_out = w0.shape[1]

    def body(d_ref, x_ref, w_ref, dinv_ref, p0_ref):
        deg = d_ref[0, :, 0:1] + d_ref[1, :, 0:1] + 1.0
        dinv = lax.rsqrt(jnp.maximum(jnp.broadcast_to(deg, (npad, 128)), 1.0))
        dinv_ref[...] = dinv
        xw = jnp.dot(x_ref[...], w_ref[...], preferred_element_type=jnp.float32)
        p0_ref[...] = xw * dinv[:, :d_out]

    return pl.pallas_call(
        body,
        out_shape=[
            jax.ShapeDtypeStruct((npad, 128), jnp.float32),
            jax.ShapeDtypeStruct((npad, d_out), jnp.float32),
        ],
    )(deg_partials, xp, w0)


def _tc_stage(s_parts, p_blocks, dinv, b2d, w_next, npad):
    nb = len(p_blocks)
    wcols = p_blocks[0].shape[1]
    d_next = w_next.shape[1]
    next_blocks = _col_blocks(d_next)

    def body(*refs):
        s_ref = refs[0]
        p_refs = refs[1:1 + nb]
        dinv_ref, b_ref, w_ref = refs[1 + nb:4 + nb]
        out_refs = refs[4 + nb:]
        dv = dinv_ref[...]
        hs = []
        for blk in range(nb):
            q = s_ref[0, blk] + s_ref[1, blk] + p_refs[blk][...]
            hb = q * dv[:, :wcols] + b_ref[0:1, blk * wcols:(blk + 1) * wcols]
            hs.append(jnp.maximum(hb, 0.0))
        h = jnp.concatenate(hs, axis=1) if nb > 1 else hs[0]
        ph = jnp.dot(h, w_ref[...], preferred_element_type=jnp.float32)
        off = 0
        for i, wn in enumerate(next_blocks):
            out_refs[i][...] = ph[:, off:off + wn] * dv[:, :wn]
            off += wn

    return pl.pallas_call(
        body,
        out_shape=[jax.ShapeDtypeStruct((npad, wn), jnp.float32)
                   for wn in next_blocks],
    )(s_parts, *p_blocks, dinv, b2d, w_next)


def _tc_final(s_parts, p_blocks, dinv, b2d, batch2d, wp1, bp1, wp2, bp2, npad):
    nb = len(p_blocks)
    wcols = p_blocks[0].shape[1]

    def body(s_ref, *refs):
        p_refs = refs[:nb]
        dinv_ref, b_ref, batch_ref, wp1_ref, bp1_ref, wp2_ref, bp2_ref, out_ref = refs[nb:]
        dv = dinv_ref[...]
        hs = []
        for blk in range(nb):
            q = s_ref[0, blk] + s_ref[1, blk] + p_refs[blk][...]
            hb = q * dv[:, :wcols] + b_ref[0:1, blk * wcols:(blk + 1) * wcols]
            hs.append(jnp.maximum(hb, 0.0))
        h = jnp.concatenate(hs, axis=1) if nb > 1 else hs[0]
        gids = lax.broadcasted_iota(jnp.int32, (_NUM_GRAPHS, npad), 0)
        bm = (jnp.broadcast_to(batch_ref[...], (_NUM_GRAPHS, npad)) == gids)
        bm = bm.astype(jnp.float32)
        sums = jnp.dot(bm, h, preferred_element_type=jnp.float32)
        counts = jnp.sum(bm, axis=1, keepdims=True)
        pooled = sums / jnp.maximum(counts, 1.0)
        z = jnp.dot(pooled, wp1_ref[...], preferred_element_type=jnp.float32)
        z = jnp.maximum(z + bp1_ref[...], 0.0)
        z = jnp.dot(z, wp2_ref[...], preferred_element_type=jnp.float32)
        z = jnp.maximum(z + bp2_ref[...], 0.0)
        m = jnp.max(z, axis=1, keepdims=True)
        e = z - m
        lse = jnp.log(jnp.sum(jnp.exp(e), axis=1, keepdims=True))
        out_ref[...] = e - lse

    return pl.pallas_call(
        body,
        out_shape=jax.ShapeDtypeStruct((_NUM_GRAPHS, 2), jnp.float32),
    )(s_parts, *p_blocks, dinv, b2d, batch2d, wp1, bp1, wp2, bp2)


def kernel(x, edge_index, batch, params):
    n = x.shape[0]
    n_edges = edge_index.shape[1]
    npad = ((n + 127) // 128) * 128
    nch = -(-n_edges // (_NW * _CHUNK))
    epad = _NW * nch * _CHUNK

    src = edge_index[0].astype(jnp.int32)
    dst = edge_index[1].astype(jnp.int32)
    pad = epad - n_edges
    src_p = jnp.concatenate([src, jnp.zeros((pad,), jnp.int32)])
    dst_p = jnp.concatenate([dst, jnp.full((pad,), n, jnp.int32)])
    src_p = src_p.reshape(_NW, nch, _CHUNK)
    dst_p = dst_p.reshape(_NW, nch, _CHUNK)

    xp = jnp.pad(x.astype(jnp.float32), ((0, npad - n), (0, 0)))
    batch2d = jnp.pad(batch.astype(jnp.int32), (0, npad - n),
                      constant_values=_NUM_GRAPHS).reshape(1, npad)

    deg_partials = _sc_degree(dst_p, npad, nch)
    dinv, p0 = _tc_init(deg_partials, xp, params['W0'], npad)

    p_blocks = [p0]
    out = None
    for i in range(9):
        s_parts = _sc_propagate(p_blocks, src_p, dst_p, npad, nch)
        b2d = params['b%d' % i].astype(jnp.float32).reshape(1, -1)
        if i < 8:
            p_blocks = _tc_stage(s_parts, p_blocks, dinv, b2d,
                                 params['W%d' % (i + 1)], npad)
        else:
            out = _tc_final(s_parts, p_blocks, dinv, b2d, batch2d,
                            params['Wp1'],
                            params['bp1'].astype(jnp.float32).reshape(1, -1),
                            params['Wp2'],
                            params['bp2'].astype(jnp.float32).reshape(1, -1),
                            npad)
    return out


# speedup vs baseline: 4.7602x; 4.7602x over previous
import functools

import jax
import jax.numpy as jnp
from jax import lax
from jax.experimental import pallas as pl
from jax.experimental.pallas import tpu as pltpu
from jax.experimental.pallas import tpu_sc as plsc

_LAYER_DIMS = [128, 64, 32, 64, 128, 64, 128, 256, 128, 256]
_NUM_GRAPHS = 64
_NC = 2
_NS = 16
_CHUNK = 128
_GRP = _NS * _CHUNK
_DEG_W = 16


def _col_blocks(d):
    out = []
    while d > 0:
        out.append(min(128, d))
        d -= 128
    return out


def _fill_f32(buf, value):
    rows, w = buf.shape
    per_row = w // 16
    vec = jnp.full((16,), value, jnp.float32)

    def body(i, _):
        r = i // per_row
        c = i % per_row
        buf[r, pl.ds(c * 16, 16)] = vec
        return 0

    lax.fori_loop(0, rows * per_row, body, 0)


def _zero_acc(acc, zbuf, s):
    nrows = acc.shape[0] - 8
    per_tile = nrows // _NS
    r0 = s * per_tile
    off = 0
    while off < per_tile:
        sz = min(_CHUNK, per_tile - off)
        pltpu.sync_copy(zbuf.at[pl.ds(0, sz)], acc.at[pl.ds(r0 + off, sz)])
        off += sz

    @pl.when(s == 0)
    def _():
        pltpu.sync_copy(zbuf.at[pl.ds(0, 8)], acc.at[pl.ds(nrows, 8)])


def _read_jmax(cnt_v):
    return cnt_v[pl.ds(0, 16)][0].astype(jnp.int32)


def _sc_degree(dstl_p, cnts, npad, nch_cap):
    split = npad // _NC
    per_tile = split // _NS
    mesh = plsc.VectorSubcoreMesh(core_axis_name="c", subcore_axis_name="s",
                                  num_cores=_NC, num_subcores=_NS)

    @functools.partial(
        pl.kernel,
        out_type=jax.ShapeDtypeStruct((npad, _DEG_W), jnp.float32),
        mesh=mesh,
        scratch_types=[
            pltpu.VMEM((nch_cap, _CHUNK), jnp.int32),
            pltpu.VMEM((_CHUNK,), jnp.float32),
            pltpu.VMEM((_CHUNK, _DEG_W), jnp.float32),
            pltpu.VMEM((_CHUNK, _DEG_W), jnp.float32),
            pltpu.VMEM_SHARED((split + 8, _DEG_W), jnp.float32),
        ],
    )
    def deg_kernel(dstl_hbm, cnt_hbm, out_hbm, idx_d, cnt_v, ones_v, zbuf, acc):
        c = lax.axis_index("c")
        s = lax.axis_index("s")
        _fill_f32(ones_v, 1.0)
        _fill_f32(zbuf, 0.0)
        _zero_acc(acc, zbuf, s)
        pltpu.sync_copy(dstl_hbm.at[c, s], idx_d)
        pltpu.sync_copy(cnt_hbm.at[c], cnt_v)
        jmax = _read_jmax(cnt_v)
        plsc.subcore_barrier()

        def body(j, _):
            pltpu.sync_copy(ones_v, acc.at[idx_d.at[j]], add=True)
            return 0

        lax.fori_loop(0, jmax, body, 0)
        plsc.subcore_barrier()
        pltpu.sync_copy(acc.at[pl.ds(s * per_tile, per_tile)],
                        out_hbm.at[pl.ds(c * split + s * per_tile, per_tile)])

    return deg_kernel(dstl_p, cnts)


def _sc_propagate(p_blocks, srcs_p, dstl_p, cnts, npad, nch_cap):
    nb = len(p_blocks)
    split = npad // _NC
    per_tile = split // _NS
    mesh = plsc.VectorSubcoreMesh(core_axis_name="c", subcore_axis_name="s",
                                  num_cores=_NC, num_subcores=_NS)

    @functools.partial(
        pl.kernel,
        out_type=jax.ShapeDtypeStruct((nb, npad, 128), jnp.float32),
        mesh=mesh,
        scratch_types=[
            pltpu.VMEM((nch_cap, _CHUNK), jnp.int32),
            pltpu.VMEM((nch_cap, _CHUNK), jnp.int32),
            pltpu.VMEM((_CHUNK,), jnp.float32),
            pltpu.VMEM((_CHUNK, 128), jnp.float32),
            pltpu.VMEM((_CHUNK, 128), jnp.float32),
            pltpu.VMEM_SHARED((split + 8, 128), jnp.float32),
            pltpu.SemaphoreType.DMA,
        ],
    )
    def prop_kernel(*refs):
        p_refs = refs[:nb]
        srcs_hbm, dstl_hbm, cnt_hbm, out_hbm = refs[nb:nb + 4]
        idx_s, idx_d, cnt_v, rows, zbuf, acc, sem = refs[nb + 4:]
        c = lax.axis_index("c")
        s = lax.axis_index("s")
        _fill_f32(zbuf, 0.0)
        pltpu.sync_copy(srcs_hbm.at[c, s], idx_s)
        pltpu.sync_copy(dstl_hbm.at[c, s], idx_d)
        pltpu.sync_copy(cnt_hbm.at[c], cnt_v)
        jmax = _read_jmax(cnt_v)
        for blk in range(nb):
            _zero_acc(acc, zbuf, s)
            plsc.subcore_barrier()

            def body(j, _, blk=blk):
                pltpu.async_copy(p_refs[blk].at[idx_s.at[j]], rows, sem).wait()
                pltpu.sync_copy(rows, acc.at[idx_d.at[j]], add=True)
                return 0

            lax.fori_loop(0, jmax, body, 0)
            plsc.subcore_barrier()
            pltpu.sync_copy(
                acc.at[pl.ds(s * per_tile, per_tile)],
                out_hbm.at[blk, pl.ds(c * split + s * per_tile, per_tile)])

    return prop_kernel(*p_blocks, srcs_p, dstl_p, cnts)


def _tc_init(deg, xp, w0, npad):
    d_out = w0.shape[1]

    def body(d_ref, x_ref, w_ref, dinv_ref, p0_ref):
        deg1 = d_ref[:, 0:1] + 1.0
        dinv = lax.rsqrt(jnp.maximum(jnp.broadcast_to(deg1, (npad, 128)), 1.0))
        dinv_ref[...] = dinv
        xw = jnp.dot(x_ref[...], w_ref[...], preferred_element_type=jnp.float32)
        val = xw * dinv[:, :d_out]
        if d_out < 128:
            val = jnp.concatenate(
                [val, jnp.zeros((npad, 128 - d_out), jnp.float32)], axis=1)
        p0_ref[...] = val

    return pl.pallas_call(
        body,
        out_shape=[
            jax.ShapeDtypeStruct((npad, 128), jnp.float32),
            jax.ShapeDtypeStruct((npad, 128), jnp.float32),
        ],
    )(deg, xp, w0)


def _tc_stage(s_parts, p_blocks, dinv, b2d, w_next, npad):
    nb = len(p_blocks)
    d_next = w_next.shape[1]
    next_blocks = _col_blocks(d_next)

    def body(*refs):
        s_ref = refs[0]
        p_refs = refs[1:1 + nb]
        dinv_ref, b_ref, w_ref = refs[1 + nb:4 + nb]
        out_refs = refs[4 + nb:]
        dv = dinv_ref[...]
        hs = []
        for blk in range(nb):
            q = s_ref[blk] + p_refs[blk][...]
            hb = q * dv + b_ref[0:1, blk * 128:(blk + 1) * 128]
            hs.append(jnp.maximum(hb, 0.0))
        h = jnp.concatenate(hs, axis=1) if nb > 1 else hs[0]
        ph = jnp.dot(h, w_ref[...], preferred_element_type=jnp.float32)
        for i, wn in enumerate(next_blocks):
            val = ph[:, i * 128:i * 128 + wn] * dv[:, :wn]
            if wn < 128:
                val = jnp.concatenate(
                    [val, jnp.zeros((npad, 128 - wn), jnp.float32)], axis=1)
            out_refs[i][...] = val

    return pl.pallas_call(
        body,
        out_shape=[jax.ShapeDtypeStruct((npad, 128), jnp.float32)
                   for _ in next_blocks],
    )(s_parts, *p_blocks, dinv, b2d, w_next)


def _tc_final(s_parts, p_blocks, dinv, b2d, batch2d, wp1, bp1, wp2, bp2, npad):
    nb = len(p_blocks)

    def body(s_ref, *refs):
        p_refs = refs[:nb]
        dinv_ref, b_ref, batch_ref, wp1_ref, bp1_ref, wp2_ref, bp2_ref, out_ref = refs[nb:]
        dv = dinv_ref[...]
        hs = []
        for blk in range(nb):
            q = s_ref[blk] + p_refs[blk][...]
            hb = q * dv + b_ref[0:1, blk * 128:(blk + 1) * 128]
            hs.append(jnp.maximum(hb, 0.0))
        h = jnp.concatenate(hs, axis=1) if nb > 1 else hs[0]
        gids = lax.broadcasted_iota(jnp.int32, (_NUM_GRAPHS, npad), 0)
        bm = (jnp.broadcast_to(batch_ref[...], (_NUM_GRAPHS, npad)) == gids)
        bm = bm.astype(jnp.float32)
        sums = jnp.dot(bm, h, preferred_element_type=jnp.float32)
        counts = jnp.sum(bm, axis=1, keepdims=True)
        pooled = sums / jnp.maximum(counts, 1.0)
        z = jnp.dot(pooled, wp1_ref[...], preferred_element_type=jnp.float32)
        z = jnp.maximum(z + bp1_ref[...], 0.0)
        z = jnp.dot(z, wp2_ref[...], preferred_element_type=jnp.float32)
        z = jnp.maximum(z + bp2_ref[...], 0.0)
        m = jnp.max(z, axis=1, keepdims=True)
        e = z - m
        lse = jnp.log(jnp.sum(jnp.exp(e), axis=1, keepdims=True))
        out_ref[...] = e - lse

    return pl.pallas_call(
        body,
        out_shape=jax.ShapeDtypeStruct((_NUM_GRAPHS, 2), jnp.float32),
    )(s_parts, *p_blocks, dinv, b2d, batch2d, wp1, bp1, wp2, bp2)


def _partition_edges(src, dst, n_edges, split, cap):
    hi = (dst >= split).astype(jnp.int32)
    pos_in_class = jnp.cumsum(hi) - 1
    lo_pos = jnp.arange(n_edges, dtype=jnp.int32) - pos_in_class - 1
    pos = jnp.where(hi == 1, pos_in_class, lo_pos)
    dummy = split
    stream_src = jnp.zeros((_NC, cap), jnp.int32).at[hi, pos].set(src)
    dst_local = dst - hi * split
    stream_dst = jnp.full((_NC, cap), dummy, jnp.int32).at[hi, pos].set(dst_local)
    cnt_hi = jnp.sum(hi)
    cnts = jnp.stack([n_edges - cnt_hi, cnt_hi])
    cnts = -(-cnts // _GRP)
    cnts = jnp.broadcast_to(cnts[:, None], (_NC, _CHUNK)).astype(jnp.float32)
    nch_cap = cap // _GRP
    stream_src = stream_src.reshape(_NC, nch_cap, _NS, _CHUNK).transpose(0, 2, 1, 3)
    stream_dst = stream_dst.reshape(_NC, nch_cap, _NS, _CHUNK).transpose(0, 2, 1, 3)
    return stream_src, stream_dst, cnts, nch_cap


def kernel(x, edge_index, batch, params):
    n = x.shape[0]
    n_edges = edge_index.shape[1]
    npad = (n // 256 + 1) * 256
    split = npad // _NC
    cap = -(-n_edges // _GRP) * _GRP

    src = edge_index[0].astype(jnp.int32)
    dst = edge_index[1].astype(jnp.int32)
    srcs_p, dstl_p, cnts, nch_cap = _partition_edges(src, dst, n_edges,
                                                     split, cap)

    xp = jnp.pad(x.astype(jnp.float32), ((0, npad - n), (0, 0)))
    batch2d = jnp.pad(batch.astype(jnp.int32), (0, npad - n),
                      constant_values=_NUM_GRAPHS).reshape(1, npad)

    deg = _sc_degree(dstl_p, cnts, npad, nch_cap)
    dinv, p0 = _tc_init(deg, xp, params['W0'], npad)

    p_blocks = [p0]
    out = None
    for i in range(9):
        d_i = _LAYER_DIMS[i + 1]
        nb = len(_col_blocks(d_i))
        s_parts = _sc_propagate(p_blocks, srcs_p, dstl_p, cnts, npad, nch_cap)
        b2d = params['b%d' % i].astype(jnp.float32).reshape(1, -1)
        b2d = jnp.pad(b2d, ((0, 0), (0, nb * 128 - d_i)))
        if i < 8:
            w_next = params['W%d' % (i + 1)].astype(jnp.float32)
            w_next = jnp.pad(w_next, ((0, nb * 128 - d_i), (0, 0)))
            p_blocks = _tc_stage(s_parts, p_blocks, dinv, b2d, w_next, npad)
        else:
            out = _tc_final(s_parts, p_blocks, dinv, b2d, batch2d,
                            params['Wp1'],
                            params['bp1'].astype(jnp.float32).reshape(1, -1),
                            params['Wp2'],
                            params['bp2'].astype(jnp.float32).reshape(1, -1),
                            npad)
    return out
